# Initial kernel scaffold; baseline (speedup 1.0000x reference)
import jax
import jax.numpy as jnp
from jax.experimental import pallas as pl


def kernel(testId, assessmentItemID, KnowledgeTag, answerCode, mask, interaction, emb_interaction, emb_test, emb_question, emb_tag, W_comb, b_comb, ln_g, ln_b):
    raise NotImplementedError("write your pallas kernel here")


# speedup vs baseline: 1.4925x; 1.4925x over previous
import functools

import jax
import jax.numpy as jnp
from jax import lax
from jax.experimental import pallas as pl
from jax.experimental.pallas import tpu as pltpu
from jax.experimental.pallas import tpu_sc as plsc

D = 1024

PAD_INT, PAD_TEST, PAD_Q, PAD_TAG = 128, 1664, 9472, 1024
OFF_TEST = PAD_INT
OFF_Q = OFF_TEST + PAD_TEST
OFF_TAG = OFF_Q + PAD_Q
R_TOTAL = OFF_TAG + PAD_TAG

BM = 128
NBLK = R_TOTAL // BM
TB1, TB2, TB3 = OFF_TEST // BM, OFF_Q // BM, OFF_TAG // BM

NC_SC, NS_SC = 2, 16
NW = NC_SC * NS_SC
CHUNK = 16


def _proj_body(a_ref, w_ref, o_ref):
    o_ref[...] = jnp.dot(a_ref[...], w_ref[...],
                         preferred_element_type=jnp.float32)


def _project_tables(t_cat, w_comb):
    def w_index(i):
        tid = ((i >= TB1).astype(jnp.int32) + (i >= TB2).astype(jnp.int32)
               + (i >= TB3).astype(jnp.int32))
        return (tid, 0)

    return pl.pallas_call(
        _proj_body,
        grid=(NBLK,),
        in_specs=[pl.BlockSpec((BM, D), lambda i: (i, 0)),
                  pl.BlockSpec((D, D), w_index)],
        out_specs=pl.BlockSpec((BM, D), lambda i: (i, 0)),
        out_shape=jax.ShapeDtypeStruct((R_TOTAL, D), jnp.float32),
    )(t_cat, w_comb)


def _make_gather_sum(rows):
    rpw = rows // NW
    nch = rpw // CHUNK
    mesh = plsc.VectorSubcoreMesh(core_axis_name="c", subcore_axis_name="s")

    @functools.partial(
        pl.kernel,
        mesh=mesh,
        out_type=jax.ShapeDtypeStruct((rows, D), jnp.float32),
        scratch_types=[
            pltpu.VMEM((4 * rpw,), jnp.int32),
            pltpu.VMEM((CHUNK, D), jnp.float32),
            pltpu.VMEM((CHUNK, D), jnp.float32),
            pltpu.VMEM((CHUNK, D), jnp.float32),
            pltpu.VMEM((CHUNK, D), jnp.float32),
            pltpu.SemaphoreType.DMA,
        ],
    )
    def gather_sum(p_hbm, idx_hbm, out_hbm, idx_v, r0, r1, r2, r3, sem):
        wid = lax.axis_index("s") * NC_SC + lax.axis_index("c")
        base = wid * rpw
        bufs = [r0, r1, r2, r3]
        for t in range(4):
            pltpu.sync_copy(idx_hbm.at[pl.ds(t * rows + base, rpw)],
                            idx_v.at[pl.ds(t * rpw, rpw)])

        def chunk_body(ci, _):
            cps = [
                pltpu.async_copy(
                    p_hbm.at[idx_v.at[pl.ds(t * rpw + ci * CHUNK, CHUNK)]],
                    bufs[t], sem)
                for t in range(4)
            ]
            for cp in cps:
                cp.wait()

            def add_row(r, _):
                def add_vec(k, _):
                    sl = pl.ds(k * 16, 16)
                    s = r0[r, sl] + r1[r, sl]
                    s = s + r2[r, sl]
                    s = s + r3[r, sl]
                    r0[r, sl] = s
                    return 0
                return lax.fori_loop(0, D // 16, add_vec, 0)

            lax.fori_loop(0, CHUNK, add_row, 0)
            pltpu.sync_copy(r0, out_hbm.at[pl.ds(base + ci * CHUNK, CHUNK)])
            return 0

        lax.fori_loop(0, nch, chunk_body, 0)

    return gather_sum


def _ln_body(x_ref, b_ref, g_ref, bb_ref, o_ref):
    x = x_ref[...] + b_ref[...]
    mu = jnp.mean(x, axis=1, keepdims=True)
    xc = x - mu
    var = jnp.mean(xc * xc, axis=1, keepdims=True)
    o_ref[...] = xc * lax.rsqrt(var + 1e-6) * g_ref[...] + bb_ref[...]


def _layernorm(ssum, b, g, bb, rows):
    bl = 512
    vec = pl.BlockSpec((1, D), lambda i: (0, 0))
    return pl.pallas_call(
        _ln_body,
        grid=(rows // bl,),
        in_specs=[pl.BlockSpec((bl, D), lambda i: (i, 0)), vec, vec, vec],
        out_specs=pl.BlockSpec((bl, D), lambda i: (i, 0)),
        out_shape=jax.ShapeDtypeStruct((rows, D), jnp.float32),
    )(ssum, b.reshape(1, D), g.reshape(1, D), bb.reshape(1, D))


def kernel(testId, assessmentItemID, KnowledgeTag, answerCode, mask,
           interaction, emb_interaction, emb_test, emb_question, emb_tag,
           W_comb, b_comb, ln_g, ln_b):
    bsz, seq = interaction.shape
    rows = bsz * seq

    def padto(x, n):
        return jnp.pad(x, ((0, n - x.shape[0]), (0, 0)))

    t_cat = jnp.concatenate([
        padto(emb_interaction, PAD_INT),
        padto(emb_test, PAD_TEST),
        padto(emb_question, PAD_Q),
        padto(emb_tag, PAD_TAG),
    ], axis=0)

    p_cat = _project_tables(t_cat, W_comb)

    idx = jnp.concatenate([
        interaction.reshape(rows),
        testId.reshape(rows) + OFF_TEST,
        assessmentItemID.reshape(rows) + OFF_Q,
        KnowledgeTag.reshape(rows) + OFF_TAG,
    ]).astype(jnp.int32)

    ssum = _make_gather_sum(rows)(p_cat, idx)
    x = _layernorm(ssum, b_comb, ln_g, ln_b, rows)
    return (x.reshape(bsz, seq, D), bsz)


# speedup vs baseline: 1.8300x; 1.2262x over previous
import functools

import jax
import jax.numpy as jnp
from jax import lax
from jax.experimental import pallas as pl
from jax.experimental.pallas import tpu as pltpu
from jax.experimental.pallas import tpu_sc as plsc

D = 1024
BM = 128

NC_SC, NS_SC = 2, 16
NW = NC_SC * NS_SC
CHUNK = 8


def _proj_body(a_ref, w_ref, o_ref):
    o_ref[...] = jnp.dot(a_ref[...], w_ref[...],
                         preferred_element_type=jnp.float32)


def _project_table(table, w_comb, t):
    v = table.shape[0]
    return pl.pallas_call(
        _proj_body,
        grid=(pl.cdiv(v, BM),),
        in_specs=[pl.BlockSpec((BM, D), lambda i: (i, 0)),
                  pl.BlockSpec((D, D), lambda i: (t, 0))],
        out_specs=pl.BlockSpec((BM, D), lambda i: (i, 0)),
        out_shape=jax.ShapeDtypeStruct((v, D), jnp.float32),
    )(table, w_comb)


def _make_gather_sum(rows):
    rpw = rows // NW
    nch = rpw // CHUNK
    mesh = plsc.VectorSubcoreMesh(core_axis_name="c", subcore_axis_name="s")
    buf_ty = pltpu.VMEM((CHUNK, D), jnp.float32)

    @functools.partial(
        pl.kernel,
        mesh=mesh,
        out_type=jax.ShapeDtypeStruct((rows, D), jnp.float32),
        scratch_types=[pltpu.VMEM((4 * rpw,), jnp.int32)]
                      + [buf_ty] * 8
                      + [pltpu.SemaphoreType.DMA, pltpu.SemaphoreType.DMA],
    )
    def gather_sum(p0, p1, p2, p3, idx_hbm, out_hbm, idx_v,
                   a0, a1, a2, a3, b0, b1, b2, b3, sem_a, sem_b):
        tabs = [p0, p1, p2, p3]
        seta, setb = [a0, a1, a2, a3], [b0, b1, b2, b3]
        wid = lax.axis_index("s") * NC_SC + lax.axis_index("c")
        base = wid * rpw
        for t in range(4):
            pltpu.sync_copy(idx_hbm.at[pl.ds(t * rows + base, rpw)],
                            idx_v.at[pl.ds(t * rpw, rpw)])

        def issue(ci, bufs, sem):
            @pl.when(ci < nch)
            def _():
                for t in range(4):
                    pltpu.async_copy(
                        tabs[t].at[idx_v.at[pl.ds(t * rpw + ci * CHUNK,
                                                  CHUNK)]],
                        bufs[t], sem)

        def wait4(bufs, sem):
            for t in range(4):
                pltpu.make_async_copy(tabs[0].at[pl.ds(0, CHUNK)],
                                      bufs[t], sem).wait()

        def add_rows(bufs):
            r0, r1, r2, r3 = bufs

            def add_row(r, _):
                for k in range(D // 16):
                    sl = pl.ds(k * 16, 16)
                    s = r0[r, sl] + r1[r, sl]
                    s = s + r2[r, sl]
                    s = s + r3[r, sl]
                    r0[r, sl] = s
                return 0

            lax.fori_loop(0, CHUNK, add_row, 0)

        def writeback(ci, bufs):
            pltpu.sync_copy(bufs[0],
                            out_hbm.at[pl.ds(base + ci * CHUNK, CHUNK)])

        issue(0, seta, sem_a)

        def pair(g, _):
            c0 = 2 * g
            issue(c0 + 1, setb, sem_b)
            wait4(seta, sem_a)
            add_rows(seta)
            writeback(c0, seta)
            issue(c0 + 2, seta, sem_a)
            wait4(setb, sem_b)
            add_rows(setb)
            writeback(c0 + 1, setb)
            return 0

        lax.fori_loop(0, nch // 2, pair, 0)

    return gather_sum


def _ln_body(x_ref, b_ref, g_ref, bb_ref, o_ref):
    x = x_ref[...] + b_ref[...]
    mu = jnp.mean(x, axis=1, keepdims=True)
    xc = x - mu
    var = jnp.mean(xc * xc, axis=1, keepdims=True)
    o_ref[...] = xc * lax.rsqrt(var + 1e-6) * g_ref[...] + bb_ref[...]


def _layernorm(ssum, b, g, bb, rows):
    bl = 512
    vec = pl.BlockSpec((1, D), lambda i: (0, 0))
    return pl.pallas_call(
        _ln_body,
        grid=(rows // bl,),
        in_specs=[pl.BlockSpec((bl, D), lambda i: (i, 0)), vec, vec, vec],
        out_specs=pl.BlockSpec((bl, D), lambda i: (i, 0)),
        out_shape=jax.ShapeDtypeStruct((rows, D), jnp.float32),
    )(ssum, b.reshape(1, D), g.reshape(1, D), bb.reshape(1, D))


def kernel(testId, assessmentItemID, KnowledgeTag, answerCode, mask,
           interaction, emb_interaction, emb_test, emb_question, emb_tag,
           W_comb, b_comb, ln_g, ln_b):
    bsz, seq = interaction.shape
    rows = bsz * seq

    p_int = _project_table(emb_interaction, W_comb, 0)
    p_test = _project_table(emb_test, W_comb, 1)
    p_q = _project_table(emb_question, W_comb, 2)
    p_tag = _project_table(emb_tag, W_comb, 3)

    idx = jnp.concatenate([
        interaction.reshape(rows),
        testId.reshape(rows),
        assessmentItemID.reshape(rows),
        KnowledgeTag.reshape(rows),
    ]).astype(jnp.int32)

    ssum = _make_gather_sum(rows)(p_int, p_test, p_q, p_tag, idx)
    x = _layernorm(ssum, b_comb, ln_g, ln_b, rows)
    return (x.reshape(bsz, seq, D), bsz)


# speedup vs baseline: 1.8855x; 1.0303x over previous
import functools

import jax
import jax.numpy as jnp
from jax import lax
from jax.experimental import pallas as pl
from jax.experimental.pallas import tpu as pltpu
from jax.experimental.pallas import tpu_sc as plsc

D = 1024

PAD_INT, PAD_TEST, PAD_Q, PAD_TAG = 128, 1664, 9472, 1024
OFF_TEST = PAD_INT
OFF_Q = OFF_TEST + PAD_TEST
OFF_TAG = OFF_Q + PAD_Q
R_TOTAL = OFF_TAG + PAD_TAG

BM = 128
NBLK = R_TOTAL // BM
TB1, TB2, TB3 = OFF_TEST // BM, OFF_Q // BM, OFF_TAG // BM

NC_SC, NS_SC = 2, 16
NW = NC_SC * NS_SC
CHUNK = 8


def _proj_body(a_ref, w_ref, o_ref):
    o_ref[...] = jnp.dot(a_ref[...], w_ref[...],
                         preferred_element_type=jnp.float32)


def _project_tables(t_cat, w_comb):
    def w_index(i):
        tid = ((i >= TB1).astype(jnp.int32) + (i >= TB2).astype(jnp.int32)
               + (i >= TB3).astype(jnp.int32))
        return (tid, 0)

    return pl.pallas_call(
        _proj_body,
        grid=(NBLK,),
        in_specs=[pl.BlockSpec((BM, D), lambda i: (i, 0)),
                  pl.BlockSpec((D, D), w_index)],
        out_specs=pl.BlockSpec((BM, D), lambda i: (i, 0)),
        out_shape=jax.ShapeDtypeStruct((R_TOTAL, D), jnp.float32),
    )(t_cat, w_comb)


def _make_gather_sum(rows):
    rpw = rows // NW
    nch = rpw // CHUNK
    gc = 4 * CHUNK
    mesh = plsc.VectorSubcoreMesh(core_axis_name="c", subcore_axis_name="s")
    buf_ty = pltpu.VMEM((gc, D), jnp.float32)

    @functools.partial(
        pl.kernel,
        mesh=mesh,
        out_type=jax.ShapeDtypeStruct((rows, D), jnp.float32),
        scratch_types=[pltpu.VMEM((4 * rpw,), jnp.int32),
                       buf_ty, buf_ty,
                       pltpu.SemaphoreType.DMA, pltpu.SemaphoreType.DMA],
    )
    def gather_sum(p_hbm, idx_hbm, out_hbm, idx_v, ga, gb, sem_a, sem_b):
        wid = lax.axis_index("s") * NC_SC + lax.axis_index("c")
        base = wid * rpw
        pltpu.sync_copy(idx_hbm.at[pl.ds(4 * base, 4 * rpw)], idx_v)

        def issue(ci, gbuf, sem):
            @pl.when(ci < nch)
            def _():
                pltpu.async_copy(
                    p_hbm.at[idx_v.at[pl.ds(ci * gc, gc)]], gbuf, sem)

        def wait(gbuf, sem):
            pltpu.make_async_copy(p_hbm.at[pl.ds(0, gc)], gbuf, sem).wait()

        def add_rows(gbuf):
            def add_row(r, _):
                for k in range(D // 16):
                    sl = pl.ds(k * 16, 16)
                    s = gbuf[r, sl] + gbuf[CHUNK + r, sl]
                    s = s + gbuf[2 * CHUNK + r, sl]
                    s = s + gbuf[3 * CHUNK + r, sl]
                    gbuf[r, sl] = s
                return 0

            lax.fori_loop(0, CHUNK, add_row, 0)

        def writeback(ci, gbuf):
            pltpu.sync_copy(gbuf.at[pl.ds(0, CHUNK)],
                            out_hbm.at[pl.ds(base + ci * CHUNK, CHUNK)])

        issue(0, ga, sem_a)

        def pair(g, _):
            c0 = 2 * g
            issue(c0 + 1, gb, sem_b)
            wait(ga, sem_a)
            add_rows(ga)
            writeback(c0, ga)
            issue(c0 + 2, ga, sem_a)
            wait(gb, sem_b)
            add_rows(gb)
            writeback(c0 + 1, gb)
            return 0

        lax.fori_loop(0, nch // 2, pair, 0)

    return gather_sum


def _make_ln_body(rb, seq):
    def _ln_body(x_ref, b_ref, g_ref, bb_ref, o_ref):
        x = x_ref[...] + b_ref[...]
        mu = jnp.mean(x, axis=1, keepdims=True)
        xc = x - mu
        var = jnp.mean(xc * xc, axis=1, keepdims=True)
        y = xc * lax.rsqrt(var + 1e-6) * g_ref[...] + bb_ref[...]
        for j in range(rb):
            o_ref[j] = y[j * seq:(j + 1) * seq, :]
    return _ln_body


def _layernorm(ssum, b, g, bb, bsz, seq):
    rb = 16
    bl = rb * seq
    vec = pl.BlockSpec((1, D), lambda i: (0, 0))
    return pl.pallas_call(
        _make_ln_body(rb, seq),
        grid=(bsz // rb,),
        in_specs=[pl.BlockSpec((bl, D), lambda i: (i, 0)), vec, vec, vec],
        out_specs=pl.BlockSpec((rb, seq, D), lambda i: (i, 0, 0)),
        out_shape=jax.ShapeDtypeStruct((bsz, seq, D), jnp.float32),
    )(ssum, b.reshape(1, D), g.reshape(1, D), bb.reshape(1, D))


def kernel(testId, assessmentItemID, KnowledgeTag, answerCode, mask,
           interaction, emb_interaction, emb_test, emb_question, emb_tag,
           W_comb, b_comb, ln_g, ln_b):
    bsz, seq = interaction.shape
    rows = bsz * seq
    rpw = rows // NW
    nch = rpw // CHUNK

    def padto(x, n):
        return jnp.pad(x, ((0, n - x.shape[0]), (0, 0)))

    t_cat = jnp.concatenate([
        padto(emb_interaction, PAD_INT),
        padto(emb_test, PAD_TEST),
        padto(emb_question, PAD_Q),
        padto(emb_tag, PAD_TAG),
    ], axis=0)

    p_cat = _project_tables(t_cat, W_comb)

    idx4 = jnp.stack([
        interaction.reshape(rows),
        testId.reshape(rows) + OFF_TEST,
        assessmentItemID.reshape(rows) + OFF_Q,
        KnowledgeTag.reshape(rows) + OFF_TAG,
    ]).astype(jnp.int32)
    idx = (idx4.reshape(4, NW, nch, CHUNK)
           .transpose(1, 2, 0, 3)
           .reshape(4 * rows))

    ssum = _make_gather_sum(rows)(p_cat, idx)
    x = _layernorm(ssum, b_comb, ln_g, ln_b, bsz, seq)
    return (x, bsz)
